# Initial kernel scaffold; baseline (speedup 1.0000x reference)
import jax
import jax.numpy as jnp
from jax.experimental import pallas as pl


def kernel(inputs, importance_embedding):
    raise NotImplementedError("write your pallas kernel here")


# speedup vs baseline: 1.1866x; 1.1866x over previous
import functools

import jax
import jax.numpy as jnp
from jax import lax
from jax.experimental import pallas as pl
from jax.experimental.pallas import tpu as pltpu
from jax.experimental.pallas import tpu_sc as plsc

B, S = 4096, 200
N = B * S
NC, NS = 2, 16
NW = NC * NS
N_PER_W = N // NW
CHUNK = 128
NCHUNK = N_PER_W // CHUNK
FIRE = 8

_mesh = plsc.VectorSubcoreMesh(core_axis_name="c", subcore_axis_name="s")


@functools.partial(
    pl.kernel,
    mesh=_mesh,
    out_type=jax.ShapeDtypeStruct((NW, NCHUNK, CHUNK), jnp.float32),
    scratch_types=[
        pltpu.VMEM((NCHUNK, CHUNK), jnp.int32),
        pltpu.VMEM((NCHUNK, CHUNK), jnp.float32),
        pltpu.SemaphoreType.DMA,
    ],
)
def _sc_gather(table_hbm, idx_hbm, out_hbm, idx_v, rows_v, sem):
    wid = lax.axis_index("s") * NC + lax.axis_index("c")
    pltpu.sync_copy(idx_hbm.at[wid], idx_v)

    def group(g, _):
        base = g * FIRE
        descs = [
            pltpu.async_copy(
                table_hbm.at[idx_v.at[base + j]], rows_v.at[base + j], sem
            )
            for j in range(FIRE)
        ]
        for d in descs:
            d.wait()
        return 0

    lax.fori_loop(0, NCHUNK // FIRE, group, 0)
    pltpu.sync_copy(rows_v, out_hbm.at[wid])


def kernel(inputs, importance_embedding):
    idx = inputs.astype(jnp.int32).reshape(NW, NCHUNK, CHUNK)
    out = _sc_gather(importance_embedding, idx)
    return out.reshape(B, S)


# speedup vs baseline: 1.5388x; 1.2968x over previous
import functools

import jax
import jax.numpy as jnp
from jax import lax
from jax.experimental import pallas as pl
from jax.experimental.pallas import tpu as pltpu
from jax.experimental.pallas import tpu_sc as plsc

B, S = 4096, 200
N = B * S
NC, NS = 2, 16
NW = NC * NS
N_PER_W = N // NW
CHUNK = 128
NCHUNK = N_PER_W // CHUNK
FIRE = 8

_mesh = plsc.VectorSubcoreMesh(core_axis_name="c", subcore_axis_name="s")


@functools.partial(
    pl.kernel,
    mesh=_mesh,
    out_type=jax.ShapeDtypeStruct((NW, N_PER_W), jnp.float32),
    scratch_types=[
        pltpu.VMEM((N_PER_W,), jnp.int32),
        pltpu.VMEM((N_PER_W,), jnp.float32),
        pltpu.SemaphoreType.DMA,
    ],
)
def _sc_gather(table_hbm, idx_hbm, out_hbm, idx_v, rows_v, sem):
    wid = lax.axis_index("s") * NC + lax.axis_index("c")
    pltpu.sync_copy(idx_hbm.at[wid], idx_v)
    pltpu.async_copy(table_hbm.at[idx_v], rows_v, sem).wait()
    pltpu.sync_copy(rows_v, out_hbm.at[wid])


def kernel(inputs, importance_embedding):
    idx = inputs.astype(jnp.int32).reshape(NW, N_PER_W)
    out = _sc_gather(importance_embedding, idx)
    return out.reshape(B, S)


# speedup vs baseline: 1.9510x; 1.2679x over previous
import functools

import jax
import jax.numpy as jnp
from jax import lax
from jax.experimental import pallas as pl
from jax.experimental.pallas import tpu as pltpu
from jax.experimental.pallas import tpu_sc as plsc

B, S = 4096, 200
N = B * S
V = 1_000_000
NC, NS = 2, 16
NW = NC * NS
N_PER_W = N // NW
STAGE = 62496
SUBSTAGE = STAGE // 3
STAGE_TAIL = V - NS * STAGE

_mesh = plsc.VectorSubcoreMesh(core_axis_name="c", subcore_axis_name="s")


@functools.partial(
    pl.kernel,
    mesh=_mesh,
    out_type=jax.ShapeDtypeStruct((NW, N_PER_W), jnp.float32),
    scratch_types=[
        pltpu.VMEM_SHARED((V,), jnp.float32),
        pltpu.VMEM((N_PER_W,), jnp.int32),
        pltpu.VMEM((N_PER_W,), jnp.float32),
        pltpu.SemaphoreType.DMA,
    ],
)
def _sc_gather(table_hbm, idx_hbm, out_hbm, table_sp, idx_v, rows_v, sem):
    cid = lax.axis_index("c")
    sid = lax.axis_index("s")
    wid = sid * NC + cid

    for s in range(3):
        pltpu.sync_copy(
            table_hbm.at[pl.ds(sid * STAGE + s * SUBSTAGE, SUBSTAGE)],
            rows_v.at[pl.ds(0, SUBSTAGE)],
        )
        pltpu.sync_copy(
            rows_v.at[pl.ds(0, SUBSTAGE)],
            table_sp.at[pl.ds(sid * STAGE + s * SUBSTAGE, SUBSTAGE)],
        )

    @pl.when(sid == 0)
    def _():
        pltpu.sync_copy(
            table_hbm.at[pl.ds(NS * STAGE, STAGE_TAIL)],
            rows_v.at[pl.ds(0, STAGE_TAIL)],
        )
        pltpu.sync_copy(
            rows_v.at[pl.ds(0, STAGE_TAIL)],
            table_sp.at[pl.ds(NS * STAGE, STAGE_TAIL)],
        )

    pltpu.sync_copy(idx_hbm.at[wid], idx_v)
    plsc.subcore_barrier()
    pltpu.async_copy(table_sp.at[idx_v], rows_v, sem).wait()
    pltpu.sync_copy(rows_v, out_hbm.at[wid])


def kernel(inputs, importance_embedding):
    idx = inputs.astype(jnp.int32).reshape(NW, N_PER_W)
    out = _sc_gather(importance_embedding, idx)
    return out.reshape(B, S)


# speedup vs baseline: 2.0692x; 1.0606x over previous
import functools

import jax
import jax.numpy as jnp
from jax import lax
from jax.experimental import pallas as pl
from jax.experimental.pallas import tpu as pltpu
from jax.experimental.pallas import tpu_sc as plsc

B, S = 4096, 200
N = B * S
V = 1_000_000
NC, NS = 2, 16
NW = NC * NS
N_PER_W = N // NW
STAGE = 62496
NSUB = 6
SUBSTAGE = STAGE // NSUB
STAGE_TAIL = V - NS * STAGE

_mesh = plsc.VectorSubcoreMesh(core_axis_name="c", subcore_axis_name="s")


@functools.partial(
    pl.kernel,
    mesh=_mesh,
    out_type=jax.ShapeDtypeStruct((NW, N_PER_W), jnp.float32),
    scratch_types=[
        pltpu.VMEM_SHARED((V,), jnp.float32),
        pltpu.VMEM((N_PER_W,), jnp.int32),
        pltpu.VMEM((N_PER_W,), jnp.float32),
        pltpu.SemaphoreType.DMA,
        pltpu.SemaphoreType.DMA,
        pltpu.SemaphoreType.DMA,
    ],
)
def _sc_gather(table_hbm, idx_hbm, out_hbm, table_sp, idx_v, rows_v, sem, isem, ssem):
    cid = lax.axis_index("c")
    sid = lax.axis_index("s")
    wid = sid * NC + cid

    idx_d = pltpu.async_copy(idx_hbm.at[wid], idx_v, isem)

    base = sid * STAGE
    loads = [
        pltpu.async_copy(
            table_hbm.at[pl.ds(base + s * SUBSTAGE, SUBSTAGE)],
            rows_v.at[pl.ds((s % 2) * SUBSTAGE, SUBSTAGE)],
            ssem,
        )
        for s in range(2)
    ]
    stores = []
    for s in range(NSUB):
        loads[s].wait()
        stores.append(
            pltpu.async_copy(
                rows_v.at[pl.ds((s % 2) * SUBSTAGE, SUBSTAGE)],
                table_sp.at[pl.ds(base + s * SUBSTAGE, SUBSTAGE)],
                sem,
            )
        )
        if s + 2 < NSUB:
            stores[s].wait()
            loads.append(
                pltpu.async_copy(
                    table_hbm.at[pl.ds(base + (s + 2) * SUBSTAGE, SUBSTAGE)],
                    rows_v.at[pl.ds((s % 2) * SUBSTAGE, SUBSTAGE)],
                    ssem,
                )
            )
    for d in stores[-2:]:
        d.wait()

    @pl.when(sid == 0)
    def _():
        pltpu.sync_copy(
            table_hbm.at[pl.ds(NS * STAGE, STAGE_TAIL)],
            rows_v.at[pl.ds(0, STAGE_TAIL)],
        )
        pltpu.sync_copy(
            rows_v.at[pl.ds(0, STAGE_TAIL)],
            table_sp.at[pl.ds(NS * STAGE, STAGE_TAIL)],
        )

    idx_d.wait()
    plsc.subcore_barrier()
    pltpu.async_copy(table_sp.at[idx_v], rows_v, sem).wait()
    pltpu.sync_copy(rows_v, out_hbm.at[wid])


def kernel(inputs, importance_embedding):
    idx = inputs.astype(jnp.int32).reshape(NW, N_PER_W)
    out = _sc_gather(importance_embedding, idx)
    return out.reshape(B, S)


# speedup vs baseline: 2.1287x; 1.0288x over previous
import functools

import jax
import jax.numpy as jnp
from jax import lax
from jax.experimental import pallas as pl
from jax.experimental.pallas import tpu as pltpu
from jax.experimental.pallas import tpu_sc as plsc

B, S = 4096, 200
N = B * S
V = 1_000_000
NC, NS = 2, 16
NW = NC * NS
N_PER_W = N // NW
STAGE = 62496
NSUB = 6
SUBSTAGE = STAGE // NSUB
STAGE_TAIL = V - NS * STAGE
KHBM = 4768

_mesh = plsc.VectorSubcoreMesh(core_axis_name="c", subcore_axis_name="s")


@functools.partial(
    pl.kernel,
    mesh=_mesh,
    out_type=jax.ShapeDtypeStruct((NW, N_PER_W), jnp.float32),
    scratch_types=[
        pltpu.VMEM_SHARED((V,), jnp.float32),
        pltpu.VMEM((N_PER_W,), jnp.int32),
        pltpu.VMEM((N_PER_W,), jnp.float32),
        pltpu.SemaphoreType.DMA,
        pltpu.SemaphoreType.DMA,
        pltpu.SemaphoreType.DMA,
    ],
)
def _sc_gather(table_hbm, idx_hbm, out_hbm, table_sp, idx_v, rows_v, sem, isem, ssem):
    cid = lax.axis_index("c")
    sid = lax.axis_index("s")
    wid = sid * NC + cid

    idx_d = pltpu.async_copy(idx_hbm.at[wid], idx_v, isem)

    base = sid * STAGE
    loads = [
        pltpu.async_copy(
            table_hbm.at[pl.ds(base + s * SUBSTAGE, SUBSTAGE)],
            rows_v.at[pl.ds((s % 2) * SUBSTAGE, SUBSTAGE)],
            ssem,
        )
        for s in range(2)
    ]
    stores = []
    for s in range(NSUB):
        loads[s].wait()
        stores.append(
            pltpu.async_copy(
                rows_v.at[pl.ds((s % 2) * SUBSTAGE, SUBSTAGE)],
                table_sp.at[pl.ds(base + s * SUBSTAGE, SUBSTAGE)],
                sem,
            )
        )
        if s + 2 < NSUB:
            stores[s].wait()
            loads.append(
                pltpu.async_copy(
                    table_hbm.at[pl.ds(base + (s + 2) * SUBSTAGE, SUBSTAGE)],
                    rows_v.at[pl.ds((s % 2) * SUBSTAGE, SUBSTAGE)],
                    ssem,
                )
            )
    for d in stores[-2:]:
        d.wait()

    @pl.when(sid == 0)
    def _():
        pltpu.sync_copy(
            table_hbm.at[pl.ds(NS * STAGE, STAGE_TAIL)],
            rows_v.at[pl.ds(0, STAGE_TAIL)],
        )
        pltpu.sync_copy(
            rows_v.at[pl.ds(0, STAGE_TAIL)],
            table_sp.at[pl.ds(NS * STAGE, STAGE_TAIL)],
        )

    idx_d.wait()
    hbm_d = pltpu.async_copy(
        table_hbm.at[idx_v.at[pl.ds(N_PER_W - KHBM, KHBM)]],
        rows_v.at[pl.ds(N_PER_W - KHBM, KHBM)],
        isem,
    )
    plsc.subcore_barrier()
    nsp = N_PER_W - KHBM
    sp_d = [
        pltpu.async_copy(
            table_sp.at[idx_v.at[pl.ds(k * (nsp // 2), nsp // 2)]],
            rows_v.at[pl.ds(k * (nsp // 2), nsp // 2)],
            sem,
        )
        for k in range(2)
    ]
    for d in sp_d:
        d.wait()
    hbm_d.wait()
    pltpu.sync_copy(rows_v, out_hbm.at[wid])


def kernel(inputs, importance_embedding):
    idx = inputs.astype(jnp.int32).reshape(NW, N_PER_W)
    out = _sc_gather(importance_embedding, idx)
    return out.reshape(B, S)
